# baseline (speedup 1.0000x reference)
import functools

import jax
import jax.numpy as jnp
from jax import lax
from jax.experimental import pallas as pl
from jax.experimental.pallas import tpu as pltpu
from jax.experimental.pallas import tpu_sc as plsc

N_NODES = 50000
N_EDGES = 800000
NUM_GRAPHS = 64

NPAD = 50176
EPAD = 851968
EROWS = EPAD // 128
NC, NS = 2, 16
SLICE = NPAD // NS

_mesh = lambda: plsc.VectorSubcoreMesh(
    core_axis_name="c", subcore_axis_name="s", num_cores=NC, num_subcores=NS)


ZCH = 448


def _fill_zero_1d(zbuf):
  def body(i, carry):
    zbuf[pl.ds(i * 16, 16)] = jnp.zeros((16,), jnp.float32)
    return carry
  lax.fori_loop(0, ZCH // 16, body, 0)


def _fill_zero_2d(zbuf, ncols):
  def body(i, carry):
    for col in range(ncols // 16):
      zbuf[i, pl.ds(col * 16, 16)] = jnp.zeros((16,), jnp.float32)
    return carry
  lax.fori_loop(0, ZCH, body, 0)


def _zero_acc(zbuf, acc, s):
  def body(k, carry):
    pltpu.sync_copy(zbuf, acc.at[pl.ds(s * SLICE + k * ZCH, ZCH)])
    return carry
  lax.fori_loop(0, SLICE // ZCH, body, 0)


def _init_acc(y_hbm, zbuf, acc, s):
  def body(k, carry):
    sl = pl.ds(s * SLICE + k * ZCH, ZCH)
    pltpu.sync_copy(y_hbm.at[sl], zbuf)
    pltpu.sync_copy(zbuf, acc.at[sl])
    return carry
  lax.fori_loop(0, SLICE // ZCH, body, 0)


def _drain_acc(acc, zbuf, out_at, s):
  def body(k, carry):
    pltpu.sync_copy(acc.at[pl.ds(s * SLICE + k * ZCH, ZCH)], zbuf)
    pltpu.sync_copy(zbuf, out_at(pl.ds(s * SLICE + k * ZCH, ZCH)))
    return carry
  lax.fori_loop(0, SLICE // ZCH, body, 0)


ECH = 16


def _edge_loop(y_hbm, src2d, dst2d, acc, bufs, tile_row0, n_chunks):
  isrc, idst, rows, sg, ss = bufs

  def chunk(i, carry):
    row0 = tile_row0 + i * ECH
    pltpu.sync_copy(src2d.at[pl.ds(row0, ECH)], isrc)
    pltpu.sync_copy(dst2d.at[pl.ds(row0, ECH)], idst)
    gd = [pltpu.async_copy(y_hbm.at[isrc.at[j]], rows.at[j], sg)
          for j in range(ECH)]
    for d in gd:
      d.wait()
    sd = [pltpu.async_copy(rows.at[j], acc.at[idst.at[j]], ss, add=True)
          for j in range(ECH)]
    for d in sd:
      d.wait()
    return carry

  lax.fori_loop(0, n_chunks, chunk, 0)


def _hist_body(dst2d, outA, outB, idx_v, ones_v, zbuf, acc, sem):
  c = lax.axis_index("c")
  s = lax.axis_index("s")
  for k in range(8):
    ones_v[pl.ds(k * 16, 16)] = jnp.full((16,), 1.0, jnp.float32)
  _fill_zero_1d(zbuf)
  _zero_acc(zbuf, acc, s)
  plsc.subcore_barrier()

  rows_per_tile = EROWS // (NC * NS)
  ch = ECH
  tile_row0 = (c * NS + s) * rows_per_tile

  def chunk(i, carry):
    row0 = tile_row0 + i * ch
    pltpu.sync_copy(dst2d.at[pl.ds(row0, ch)], idx_v)
    descs = []
    for j in range(ch):
      descs.append(
          pltpu.async_copy(ones_v, acc.at[idx_v.at[j]], sem, add=True))
    for d in descs:
      d.wait()
    return carry

  lax.fori_loop(0, rows_per_tile // ch, chunk, 0)
  plsc.subcore_barrier()

  @pl.when(c == 0)
  def _():
    _drain_acc(acc, zbuf, lambda sl: outA.at[sl], s)

  @pl.when(c == 1)
  def _():
    _drain_acc(acc, zbuf, lambda sl: outB.at[sl], s)


def _hist(dst2d):
  return pl.kernel(
      _hist_body,
      out_type=[
          jax.ShapeDtypeStruct((NPAD,), jnp.float32),
          jax.ShapeDtypeStruct((NPAD,), jnp.float32),
      ],
      mesh=_mesh(),
      scratch_types=[
          pltpu.VMEM((ECH, 128), jnp.int32),
          pltpu.VMEM((128,), jnp.float32),
          pltpu.VMEM((ZCH,), jnp.float32),
          pltpu.VMEM_SHARED((NPAD,), jnp.float32),
          pltpu.SemaphoreType.DMA,
      ],
  )(dst2d)


def _prop1_body(y1, src2d, dst2d, out, isrc, idst, rows, zbuf, acc, sg, ss):
  c = lax.axis_index("c")
  s = lax.axis_index("s")

  @pl.when(c == 0)
  def _():
    _init_acc(y1, zbuf, acc, s)

  @pl.when(c == 1)
  def _():
    _fill_zero_2d(zbuf, 16)
    _zero_acc(zbuf, acc, s)

  plsc.subcore_barrier()

  rows_per_tile = EROWS // (NC * NS)
  tile_row0 = (c * NS + s) * rows_per_tile
  bufs = (isrc, idst, rows, sg, ss)
  _edge_loop(y1, src2d, dst2d, acc, bufs, tile_row0, rows_per_tile // ECH)

  plsc.subcore_barrier()
  _drain_acc(acc, zbuf, lambda sl: out.at[c, sl], s)


_PROP_SCRATCH16 = [
    pltpu.VMEM((ECH, 128), jnp.int32),
    pltpu.VMEM((ECH, 128), jnp.int32),
    pltpu.VMEM((ECH, 128, 16), jnp.float32),
    pltpu.VMEM((ZCH, 16), jnp.float32),
    pltpu.VMEM_SHARED((NPAD, 16), jnp.float32),
    pltpu.SemaphoreType.DMA,
    pltpu.SemaphoreType.DMA,
]


def _prop1(y1, src2d, dst2d):
  return pl.kernel(
      _prop1_body,
      out_type=jax.ShapeDtypeStruct((NC, NPAD, 16), jnp.float32),
      mesh=_mesh(),
      scratch_types=_PROP_SCRATCH16,
      compiler_params=pltpu.CompilerParams(use_tc_tiling_on_sc=False),
  )(y1, src2d, dst2d)


def _prop2_body(y0, y1, y2, y3, src2d, dst2d, out0, out1, out2, out3,
                isrc, idst, rows, zbuf, acc, sg, ss):
  c = lax.axis_index("c")
  s = lax.axis_index("s")
  rows_per_tile = EROWS // NS
  tile_row0 = s * rows_per_tile
  bufs = (isrc, idst, rows, sg, ss)

  def phase(y_hbm, out_hbm):
    _init_acc(y_hbm, zbuf, acc, s)
    plsc.subcore_barrier()
    _edge_loop(y_hbm, src2d, dst2d, acc, bufs, tile_row0,
               rows_per_tile // ECH)
    plsc.subcore_barrier()
    _drain_acc(acc, zbuf, lambda sl: out_hbm.at[sl], s)
    plsc.subcore_barrier()

  @pl.when(c == 0)
  def _():
    phase(y0, out0)
    phase(y1, out1)

  @pl.when(c == 1)
  def _():
    phase(y2, out2)
    phase(y3, out3)


def _prop2(yq, src2d, dst2d):
  return pl.kernel(
      _prop2_body,
      out_type=[jax.ShapeDtypeStruct((NPAD, 16), jnp.float32)] * 4,
      mesh=_mesh(),
      scratch_types=_PROP_SCRATCH16,
      compiler_params=pltpu.CompilerParams(use_tc_tiling_on_sc=False),
  )(yq[0], yq[1], yq[2], yq[3], src2d, dst2d)


RT1 = 6272
RT = 3584


def _prep_body(deg_ref, x_ref, dis_ref, y1_ref):
  deg = deg_ref[0] + deg_ref[1] + 1.0
  dis = lax.rsqrt(deg)
  dis_ref[...] = dis
  y1_ref[...] = x_ref[...] * dis


def _tc_prep(degs3, x_p):
  return pl.pallas_call(
      _prep_body,
      grid=(NPAD // RT1,),
      in_specs=[
          pl.BlockSpec((NC, RT1, 1), lambda i: (0, i, 0)),
          pl.BlockSpec((RT1, 16), lambda i: (i, 0)),
      ],
      out_specs=[
          pl.BlockSpec((RT1, 1), lambda i: (i, 0)),
          pl.BlockSpec((RT1, 16), lambda i: (i, 0)),
      ],
      out_shape=[
          jax.ShapeDtypeStruct((NPAD, 1), jnp.float32),
          jax.ShapeDtypeStruct((NPAD, 16), jnp.float32),
      ],
  )(degs3, x_p)


def _layer1_body(dis_ref, z1_ref, w_ref, b_ref, *yq_refs):
  dis = dis_ref[...]
  p1 = (z1_ref[0] + z1_ref[1]) * dis
  h1 = jnp.maximum(
      jnp.dot(p1, w_ref[...], preferred_element_type=jnp.float32)
      + b_ref[...], 0.0)
  y2 = h1 * dis
  for q in range(4):
    yq_refs[q][...] = y2[:, q * 16:(q + 1) * 16]


def _tc_layer1(dis, z1, w1p, b1):
  return pl.pallas_call(
      _layer1_body,
      grid=(NPAD // RT,),
      in_specs=[
          pl.BlockSpec((RT, 1), lambda i: (i, 0)),
          pl.BlockSpec((NC, RT, 16), lambda i: (0, i, 0)),
          pl.BlockSpec((16, 64), lambda i: (0, 0)),
          pl.BlockSpec((1, 64), lambda i: (0, 0)),
      ],
      out_specs=[pl.BlockSpec((RT, 16), lambda i: (i, 0))] * 4,
      out_shape=[jax.ShapeDtypeStruct((NPAD, 16), jnp.float32)] * 4,
  )(dis, z1, w1p, b1)


def _layer2_body(dis_ref, z0_ref, z1_ref, z2_ref, z3_ref, batch_ref,
                 w2_ref, b2_ref, wf1_ref, bf1_ref, wf2_ref, bf2_ref,
                 out_ref, sums_ref, counts_ref):
  i = pl.program_id(0)
  nblk = pl.num_programs(0)
  dis = dis_ref[...]
  p2 = jnp.concatenate(
      [z_ref[...] * dis
       for z_ref in [z0_ref, z1_ref, z2_ref, z3_ref]],
      axis=1)
  h2 = jnp.maximum(
      jnp.dot(p2, w2_ref[...], preferred_element_type=jnp.float32)
      + b2_ref[...], 0.0)
  gid = lax.broadcasted_iota(jnp.int32, (RT, NUM_GRAPHS), 1)
  oh = (batch_ref[...] == gid).astype(jnp.float32)
  blk_sums = lax.dot_general(oh, h2, (((0,), (0,)), ((), ())),
                             preferred_element_type=jnp.float32)
  ones = jnp.ones((RT, 1), jnp.float32)
  blk_counts = lax.dot_general(oh, ones, (((0,), (0,)), ((), ())),
                               preferred_element_type=jnp.float32)

  @pl.when(i == 0)
  def _():
    sums_ref[...] = blk_sums
    counts_ref[...] = blk_counts

  @pl.when(i > 0)
  def _():
    sums_ref[...] += blk_sums
    counts_ref[...] += blk_counts

  @pl.when(i == nblk - 1)
  def _():
    g = sums_ref[...] / jnp.maximum(counts_ref[...], 1.0)
    o1 = jnp.maximum(
        jnp.dot(g, wf1_ref[...], preferred_element_type=jnp.float32)
        + bf1_ref[...], 0.0)
    out_ref[...] = (
        jnp.dot(o1, wf2_ref[...], preferred_element_type=jnp.float32)
        + bf2_ref[...])


def _tc_layer2(dis, zq, batch2d, w2, b2, wf1, bf1, wf2, bf2):
  row = lambda i: (i, 0)
  full = lambda i: (0, 0)
  return pl.pallas_call(
      _layer2_body,
      grid=(NPAD // RT,),
      in_specs=[
          pl.BlockSpec((RT, 1), row),
          pl.BlockSpec((RT, 16), row),
          pl.BlockSpec((RT, 16), row),
          pl.BlockSpec((RT, 16), row),
          pl.BlockSpec((RT, 16), row),
          pl.BlockSpec((RT, 1), row),
          pl.BlockSpec((64, 128), full),
          pl.BlockSpec((1, 128), full),
          pl.BlockSpec((128, 64), full),
          pl.BlockSpec((1, 64), full),
          pl.BlockSpec((64, 1), full),
          pl.BlockSpec((1, 1), full),
      ],
      out_specs=pl.BlockSpec((NUM_GRAPHS, 1), full),
      out_shape=jax.ShapeDtypeStruct((NUM_GRAPHS, 1), jnp.float32),
      scratch_shapes=[
          pltpu.VMEM((NUM_GRAPHS, 128), jnp.float32),
          pltpu.VMEM((NUM_GRAPHS, 1), jnp.float32),
      ],
  )(dis, *zq, batch2d, w2, b2, wf1, bf1, wf2, bf2)


@jax.jit
def kernel(x, edge_index, batch, W1, b1, W2, b2, Wf1, bf1, Wf2, bf2):
  src = edge_index[0].astype(jnp.int32)
  dst = edge_index[1].astype(jnp.int32)
  epad = N_NODES + jnp.arange(EPAD - N_EDGES, dtype=jnp.int32) % (NPAD - N_NODES)
  src2d = jnp.concatenate([src, epad]).reshape(EROWS, 128)
  dst2d = jnp.concatenate([dst, epad]).reshape(EROWS, 128)
  batch2d = jnp.concatenate(
      [batch.astype(jnp.int32),
       jnp.full((NPAD - N_NODES,), NUM_GRAPHS, jnp.int32)]).reshape(NPAD, 1)
  x_p = jnp.zeros((NPAD, 16), jnp.float32).at[:N_NODES, :3].set(x)
  w1p = jnp.zeros((16, 64), jnp.float32).at[:3].set(W1)

  degA, degB = _hist(dst2d)
  dis, y1 = _tc_prep(jnp.stack([degA, degB]).reshape(NC, NPAD, 1), x_p)
  z1 = _prop1(y1, src2d, dst2d)
  yq = _tc_layer1(dis, z1, w1p, b1.reshape(1, 64))
  zq = _prop2(yq, src2d, dst2d)
  out = _tc_layer2(dis, zq, batch2d,
                   W2, b2.reshape(1, 128), Wf1, bf1.reshape(1, 64),
                   Wf2, bf2.reshape(1, 1))
  return out


# speedup vs baseline: 1.0537x; 1.0537x over previous
import functools

import jax
import jax.numpy as jnp
from jax import lax
from jax.experimental import pallas as pl
from jax.experimental.pallas import tpu as pltpu
from jax.experimental.pallas import tpu_sc as plsc

N_NODES = 50000
N_EDGES = 800000
NUM_GRAPHS = 64

NPAD = 50176
EPAD = 851968
EROWS = EPAD // 128
NC, NS = 2, 16
SLICE = NPAD // NS

_mesh = lambda: plsc.VectorSubcoreMesh(
    core_axis_name="c", subcore_axis_name="s", num_cores=NC, num_subcores=NS)


ZCH = 448


def _fill_zero_1d(zbuf):
  def body(i, carry):
    zbuf[pl.ds(i * 16, 16)] = jnp.zeros((16,), jnp.float32)
    return carry
  lax.fori_loop(0, ZCH // 16, body, 0)


def _fill_zero_2d(zbuf, ncols):
  def body(i, carry):
    for col in range(ncols // 16):
      zbuf[i, pl.ds(col * 16, 16)] = jnp.zeros((16,), jnp.float32)
    return carry
  lax.fori_loop(0, ZCH, body, 0)


def _zero_acc(zbuf, acc, s):
  def body(k, carry):
    pltpu.sync_copy(zbuf, acc.at[pl.ds(s * SLICE + k * ZCH, ZCH)])
    return carry
  lax.fori_loop(0, SLICE // ZCH, body, 0)


def _init_acc(y_hbm, zbuf, acc, s):
  def body(k, carry):
    sl = pl.ds(s * SLICE + k * ZCH, ZCH)
    pltpu.sync_copy(y_hbm.at[sl], zbuf)
    pltpu.sync_copy(zbuf, acc.at[sl])
    return carry
  lax.fori_loop(0, SLICE // ZCH, body, 0)


def _drain_acc(acc, zbuf, out_at, s):
  def body(k, carry):
    pltpu.sync_copy(acc.at[pl.ds(s * SLICE + k * ZCH, ZCH)], zbuf)
    pltpu.sync_copy(zbuf, out_at(pl.ds(s * SLICE + k * ZCH, ZCH)))
    return carry
  lax.fori_loop(0, SLICE // ZCH, body, 0)


ECH = 16


def _edge_loop(y_hbm, src2d, dst2d, acc, bufs, tile_row0, n_chunks):
  isrc, idst, rows, sg, ss = bufs

  def chunk(i, carry):
    row0 = tile_row0 + i * ECH
    pltpu.sync_copy(src2d.at[pl.ds(row0, ECH)], isrc)
    pltpu.sync_copy(dst2d.at[pl.ds(row0, ECH)], idst)
    gd = [pltpu.async_copy(y_hbm.at[isrc.at[j]], rows.at[j], sg)
          for j in range(ECH)]
    for d in gd:
      d.wait()
    sd = [pltpu.async_copy(rows.at[j], acc.at[idst.at[j]], ss, add=True)
          for j in range(ECH)]
    for d in sd:
      d.wait()
    return carry

  lax.fori_loop(0, n_chunks, chunk, 0)


def _hist_body(dst2d, outA, outB, idx_v, ones_v, zbuf, acc, sem):
  c = lax.axis_index("c")
  s = lax.axis_index("s")
  for k in range(8):
    ones_v[pl.ds(k * 16, 16)] = jnp.full((16,), 1.0, jnp.float32)
  _fill_zero_1d(zbuf)
  _zero_acc(zbuf, acc, s)
  plsc.subcore_barrier()

  rows_per_tile = EROWS // (NC * NS)
  ch = ECH
  tile_row0 = (c * NS + s) * rows_per_tile

  def chunk(i, carry):
    row0 = tile_row0 + i * ch
    pltpu.sync_copy(dst2d.at[pl.ds(row0, ch)], idx_v)
    descs = []
    for j in range(ch):
      descs.append(
          pltpu.async_copy(ones_v, acc.at[idx_v.at[j]], sem, add=True))
    for d in descs:
      d.wait()
    return carry

  lax.fori_loop(0, rows_per_tile // ch, chunk, 0)
  plsc.subcore_barrier()

  @pl.when(c == 0)
  def _():
    _drain_acc(acc, zbuf, lambda sl: outA.at[sl], s)

  @pl.when(c == 1)
  def _():
    _drain_acc(acc, zbuf, lambda sl: outB.at[sl], s)


def _hist(dst2d):
  return pl.kernel(
      _hist_body,
      out_type=[
          jax.ShapeDtypeStruct((NPAD,), jnp.float32),
          jax.ShapeDtypeStruct((NPAD,), jnp.float32),
      ],
      mesh=_mesh(),
      scratch_types=[
          pltpu.VMEM((ECH, 128), jnp.int32),
          pltpu.VMEM((128,), jnp.float32),
          pltpu.VMEM((ZCH,), jnp.float32),
          pltpu.VMEM_SHARED((NPAD,), jnp.float32),
          pltpu.SemaphoreType.DMA,
      ],
  )(dst2d)


def _prop1_body(y1, src2d, dst2d, out, isrc, idst, rows, zbuf, acc, sg, ss):
  c = lax.axis_index("c")
  s = lax.axis_index("s")

  @pl.when(c == 0)
  def _():
    _init_acc(y1, zbuf, acc, s)

  @pl.when(c == 1)
  def _():
    _fill_zero_2d(zbuf, 16)
    _zero_acc(zbuf, acc, s)

  plsc.subcore_barrier()

  rows_per_tile = EROWS // (NC * NS)
  tile_row0 = (c * NS + s) * rows_per_tile
  bufs = (isrc, idst, rows, sg, ss)
  _edge_loop(y1, src2d, dst2d, acc, bufs, tile_row0, rows_per_tile // ECH)

  plsc.subcore_barrier()
  _drain_acc(acc, zbuf, lambda sl: out.at[c, sl], s)


_PROP_SCRATCH16 = [
    pltpu.VMEM((ECH, 128), jnp.int32),
    pltpu.VMEM((ECH, 128), jnp.int32),
    pltpu.VMEM((ECH, 128, 16), jnp.float32),
    pltpu.VMEM((ZCH, 16), jnp.float32),
    pltpu.VMEM_SHARED((NPAD, 16), jnp.float32),
    pltpu.SemaphoreType.DMA,
    pltpu.SemaphoreType.DMA,
]


def _prop1(y1, src2d, dst2d):
  return pl.kernel(
      _prop1_body,
      out_type=jax.ShapeDtypeStruct((NC, NPAD, 16), jnp.float32),
      mesh=_mesh(),
      scratch_types=_PROP_SCRATCH16,
      compiler_params=pltpu.CompilerParams(use_tc_tiling_on_sc=False),
  )(y1, src2d, dst2d)


def _prop2_body(ya, yb, src2d, dst2d, outa, outb,
                isrc, idst, rows, zbuf, acc, sg, ss):
  c = lax.axis_index("c")
  s = lax.axis_index("s")
  rows_per_tile = EROWS // NS
  tile_row0 = s * rows_per_tile
  bufs = (isrc, idst, rows, sg, ss)

  def phase(y_hbm, out_hbm):
    _init_acc(y_hbm, zbuf, acc, s)
    plsc.subcore_barrier()
    _edge_loop(y_hbm, src2d, dst2d, acc, bufs, tile_row0,
               rows_per_tile // ECH)
    plsc.subcore_barrier()
    _drain_acc(acc, zbuf, lambda sl: out_hbm.at[sl], s)

  @pl.when(c == 0)
  def _():
    phase(ya, outa)

  @pl.when(c == 1)
  def _():
    phase(yb, outb)


def _prop2(ya, yb, src2d, dst2d):
  return pl.kernel(
      _prop2_body,
      out_type=[jax.ShapeDtypeStruct((NPAD, 16), jnp.float32)] * 2,
      mesh=_mesh(),
      scratch_types=_PROP_SCRATCH16,
      compiler_params=pltpu.CompilerParams(use_tc_tiling_on_sc=False),
  )(ya, yb, src2d, dst2d)


RT1 = 6272
RT = 3584


def _prep_body(deg_ref, x_ref, dis_ref, y1_ref):
  deg = deg_ref[0] + deg_ref[1] + 1.0
  dis = lax.rsqrt(deg)
  dis_ref[...] = dis
  y1_ref[...] = x_ref[...] * dis


def _tc_prep(degs3, x_p):
  return pl.pallas_call(
      _prep_body,
      grid=(NPAD // RT1,),
      in_specs=[
          pl.BlockSpec((NC, RT1, 1), lambda i: (0, i, 0)),
          pl.BlockSpec((RT1, 16), lambda i: (i, 0)),
      ],
      out_specs=[
          pl.BlockSpec((RT1, 1), lambda i: (i, 0)),
          pl.BlockSpec((RT1, 16), lambda i: (i, 0)),
      ],
      out_shape=[
          jax.ShapeDtypeStruct((NPAD, 1), jnp.float32),
          jax.ShapeDtypeStruct((NPAD, 16), jnp.float32),
      ],
  )(degs3, x_p)


def _layer1_body(dis_ref, z1_ref, w_ref, b_ref, *yq_refs):
  dis = dis_ref[...]
  p1 = (z1_ref[0] + z1_ref[1]) * dis
  h1 = jnp.maximum(
      jnp.dot(p1, w_ref[...], preferred_element_type=jnp.float32)
      + b_ref[...], 0.0)
  y2 = h1 * dis
  for q in range(4):
    yq_refs[q][...] = y2[:, q * 16:(q + 1) * 16]


def _tc_layer1(dis, z1, w1p, b1):
  return pl.pallas_call(
      _layer1_body,
      grid=(NPAD // RT,),
      in_specs=[
          pl.BlockSpec((RT, 1), lambda i: (i, 0)),
          pl.BlockSpec((NC, RT, 16), lambda i: (0, i, 0)),
          pl.BlockSpec((16, 64), lambda i: (0, 0)),
          pl.BlockSpec((1, 64), lambda i: (0, 0)),
      ],
      out_specs=[pl.BlockSpec((RT, 16), lambda i: (i, 0))] * 4,
      out_shape=[jax.ShapeDtypeStruct((NPAD, 16), jnp.float32)] * 4,
  )(dis, z1, w1p, b1)


def _layer2_body(dis_ref, z0_ref, z1_ref, z2_ref, z3_ref, batch_ref,
                 w2_ref, b2_ref, wf1_ref, bf1_ref, wf2_ref, bf2_ref,
                 out_ref, sums_ref, counts_ref):
  i = pl.program_id(0)
  nblk = pl.num_programs(0)
  dis = dis_ref[...]
  p2 = jnp.concatenate(
      [z_ref[...] * dis
       for z_ref in [z0_ref, z1_ref, z2_ref, z3_ref]],
      axis=1)
  h2 = jnp.maximum(
      jnp.dot(p2, w2_ref[...], preferred_element_type=jnp.float32)
      + b2_ref[...], 0.0)
  gid = lax.broadcasted_iota(jnp.int32, (RT, NUM_GRAPHS), 1)
  oh = (batch_ref[...] == gid).astype(jnp.float32)
  blk_sums = lax.dot_general(oh, h2, (((0,), (0,)), ((), ())),
                             preferred_element_type=jnp.float32)
  ones = jnp.ones((RT, 1), jnp.float32)
  blk_counts = lax.dot_general(oh, ones, (((0,), (0,)), ((), ())),
                               preferred_element_type=jnp.float32)

  @pl.when(i == 0)
  def _():
    sums_ref[...] = blk_sums
    counts_ref[...] = blk_counts

  @pl.when(i > 0)
  def _():
    sums_ref[...] += blk_sums
    counts_ref[...] += blk_counts

  @pl.when(i == nblk - 1)
  def _():
    g = sums_ref[...] / jnp.maximum(counts_ref[...], 1.0)
    o1 = jnp.maximum(
        jnp.dot(g, wf1_ref[...], preferred_element_type=jnp.float32)
        + bf1_ref[...], 0.0)
    out_ref[...] = (
        jnp.dot(o1, wf2_ref[...], preferred_element_type=jnp.float32)
        + bf2_ref[...])


def _tc_layer2(dis, zq, batch2d, w2, b2, wf1, bf1, wf2, bf2):
  row = lambda i: (i, 0)
  full = lambda i: (0, 0)
  return pl.pallas_call(
      _layer2_body,
      grid=(NPAD // RT,),
      in_specs=[
          pl.BlockSpec((RT, 1), row),
          pl.BlockSpec((RT, 16), row),
          pl.BlockSpec((RT, 16), row),
          pl.BlockSpec((RT, 16), row),
          pl.BlockSpec((RT, 16), row),
          pl.BlockSpec((RT, 1), row),
          pl.BlockSpec((64, 128), full),
          pl.BlockSpec((1, 128), full),
          pl.BlockSpec((128, 64), full),
          pl.BlockSpec((1, 64), full),
          pl.BlockSpec((64, 1), full),
          pl.BlockSpec((1, 1), full),
      ],
      out_specs=pl.BlockSpec((NUM_GRAPHS, 1), full),
      out_shape=jax.ShapeDtypeStruct((NUM_GRAPHS, 1), jnp.float32),
      scratch_shapes=[
          pltpu.VMEM((NUM_GRAPHS, 128), jnp.float32),
          pltpu.VMEM((NUM_GRAPHS, 1), jnp.float32),
      ],
  )(dis, *zq, batch2d, w2, b2, wf1, bf1, wf2, bf2)


@jax.jit
def kernel(x, edge_index, batch, W1, b1, W2, b2, Wf1, bf1, Wf2, bf2):
  src = edge_index[0].astype(jnp.int32)
  dst = edge_index[1].astype(jnp.int32)
  epad = N_NODES + jnp.arange(EPAD - N_EDGES, dtype=jnp.int32) % (NPAD - N_NODES)
  src2d = jnp.concatenate([src, epad]).reshape(EROWS, 128)
  dst2d = jnp.concatenate([dst, epad]).reshape(EROWS, 128)
  batch2d = jnp.concatenate(
      [batch.astype(jnp.int32),
       jnp.full((NPAD - N_NODES,), NUM_GRAPHS, jnp.int32)]).reshape(NPAD, 1)
  x_p = jnp.zeros((NPAD, 16), jnp.float32).at[:N_NODES, :3].set(x)
  w1p = jnp.zeros((16, 64), jnp.float32).at[:3].set(W1)

  degA, degB = _hist(dst2d)
  dis, y1 = _tc_prep(jnp.stack([degA, degB]).reshape(NC, NPAD, 1), x_p)
  z1 = _prop1(y1, src2d, dst2d)
  yq = _tc_layer1(dis, z1, w1p, b1.reshape(1, 64))
  z0, z1b = _prop2(yq[0], yq[1], src2d, dst2d)
  z2, z3 = _prop2(yq[2], yq[3], src2d, dst2d)
  out = _tc_layer2(dis, (z0, z1b, z2, z3), batch2d,
                   W2, b2.reshape(1, 128), Wf1, bf1.reshape(1, 64),
                   Wf2, bf2.reshape(1, 1))
  return out


# speedup vs baseline: 1.0567x; 1.0029x over previous
import functools

import jax
import jax.numpy as jnp
from jax import lax
from jax.experimental import pallas as pl
from jax.experimental.pallas import tpu as pltpu
from jax.experimental.pallas import tpu_sc as plsc

N_NODES = 50000
N_EDGES = 800000
NUM_GRAPHS = 64

NPAD = 50176
EPAD = 851968
EROWS = EPAD // 128
NC, NS = 2, 16
SLICE = NPAD // NS

_mesh = lambda: plsc.VectorSubcoreMesh(
    core_axis_name="c", subcore_axis_name="s", num_cores=NC, num_subcores=NS)


ZCH = 448


def _fill_zero_1d(zbuf):
  def body(i, carry):
    zbuf[pl.ds(i * 16, 16)] = jnp.zeros((16,), jnp.float32)
    return carry
  lax.fori_loop(0, ZCH // 16, body, 0)


def _fill_zero_2d(zbuf, ncols):
  def body(i, carry):
    for col in range(ncols // 16):
      zbuf[i, pl.ds(col * 16, 16)] = jnp.zeros((16,), jnp.float32)
    return carry
  lax.fori_loop(0, ZCH, body, 0)


def _zero_acc(zbuf, acc, s):
  def body(k, carry):
    pltpu.sync_copy(zbuf, acc.at[pl.ds(s * SLICE + k * ZCH, ZCH)])
    return carry
  lax.fori_loop(0, SLICE // ZCH, body, 0)


def _init_acc(y_hbm, zbuf, acc, s):
  def body(k, carry):
    sl = pl.ds(s * SLICE + k * ZCH, ZCH)
    pltpu.sync_copy(y_hbm.at[sl], zbuf)
    pltpu.sync_copy(zbuf, acc.at[sl])
    return carry
  lax.fori_loop(0, SLICE // ZCH, body, 0)


def _drain_acc(acc, zbuf, out_at, s):
  def body(k, carry):
    pltpu.sync_copy(acc.at[pl.ds(s * SLICE + k * ZCH, ZCH)], zbuf)
    pltpu.sync_copy(zbuf, out_at(pl.ds(s * SLICE + k * ZCH, ZCH)))
    return carry
  lax.fori_loop(0, SLICE // ZCH, body, 0)


ECH = 16


def _edge_loop(y_hbm, src2d, dst2d, acc, bufs, tile_row0, n_chunks):
  isrc, idst, rows, sg, ss = bufs

  def chunk(i, carry):
    row0 = tile_row0 + i * ECH
    pltpu.sync_copy(src2d.at[pl.ds(row0, ECH)], isrc)
    pltpu.sync_copy(dst2d.at[pl.ds(row0, ECH)], idst)
    gd = [pltpu.async_copy(y_hbm.at[isrc.at[j]], rows.at[j], sg)
          for j in range(ECH)]
    for d in gd:
      d.wait()
    sd = [pltpu.async_copy(rows.at[j], acc.at[idst.at[j]], ss, add=True)
          for j in range(ECH)]
    for d in sd:
      d.wait()
    return carry

  lax.fori_loop(0, n_chunks, chunk, 0)


def _hist_body(dst2d, outA, outB, idx_v, ones_v, zbuf, acc, sem):
  c = lax.axis_index("c")
  s = lax.axis_index("s")
  for k in range(8):
    ones_v[pl.ds(k * 16, 16)] = jnp.full((16,), 1.0, jnp.float32)
  _fill_zero_1d(zbuf)
  _zero_acc(zbuf, acc, s)
  plsc.subcore_barrier()

  rows_per_tile = EROWS // (NC * NS)
  ch = ECH
  tile_row0 = (c * NS + s) * rows_per_tile

  def chunk(i, carry):
    row0 = tile_row0 + i * ch
    pltpu.sync_copy(dst2d.at[pl.ds(row0, ch)], idx_v)
    descs = []
    for j in range(ch):
      descs.append(
          pltpu.async_copy(ones_v, acc.at[idx_v.at[j]], sem, add=True))
    for d in descs:
      d.wait()
    return carry

  lax.fori_loop(0, rows_per_tile // ch, chunk, 0)
  plsc.subcore_barrier()

  @pl.when(c == 0)
  def _():
    _drain_acc(acc, zbuf, lambda sl: outA.at[sl], s)

  @pl.when(c == 1)
  def _():
    _drain_acc(acc, zbuf, lambda sl: outB.at[sl], s)


def _hist(dst2d):
  return pl.kernel(
      _hist_body,
      out_type=[
          jax.ShapeDtypeStruct((NPAD,), jnp.float32),
          jax.ShapeDtypeStruct((NPAD,), jnp.float32),
      ],
      mesh=_mesh(),
      scratch_types=[
          pltpu.VMEM((ECH, 128), jnp.int32),
          pltpu.VMEM((128,), jnp.float32),
          pltpu.VMEM((ZCH,), jnp.float32),
          pltpu.VMEM_SHARED((NPAD,), jnp.float32),
          pltpu.SemaphoreType.DMA,
      ],
  )(dst2d)


def _prop1_body(y1, src2d, dst2d, out, isrc, idst, rows, zbuf, acc, sg, ss):
  c = lax.axis_index("c")
  s = lax.axis_index("s")

  @pl.when(c == 0)
  def _():
    _init_acc(y1, zbuf, acc, s)

  @pl.when(c == 1)
  def _():
    _fill_zero_2d(zbuf, 16)
    _zero_acc(zbuf, acc, s)

  plsc.subcore_barrier()

  rows_per_tile = EROWS // (NC * NS)
  tile_row0 = (c * NS + s) * rows_per_tile
  bufs = (isrc, idst, rows, sg, ss)
  _edge_loop(y1, src2d, dst2d, acc, bufs, tile_row0, rows_per_tile // ECH)

  plsc.subcore_barrier()
  _drain_acc(acc, zbuf, lambda sl: out.at[c, sl], s)


_PROP_SCRATCH16 = [
    pltpu.VMEM((ECH, 128), jnp.int32),
    pltpu.VMEM((ECH, 128), jnp.int32),
    pltpu.VMEM((ECH, 128, 16), jnp.float32),
    pltpu.VMEM((ZCH, 16), jnp.float32),
    pltpu.VMEM_SHARED((NPAD, 16), jnp.float32),
    pltpu.SemaphoreType.DMA,
    pltpu.SemaphoreType.DMA,
]


def _prop1(y1, src2d, dst2d):
  return pl.kernel(
      _prop1_body,
      out_type=jax.ShapeDtypeStruct((NC, NPAD, 16), jnp.float32),
      mesh=_mesh(),
      scratch_types=_PROP_SCRATCH16,
      compiler_params=pltpu.CompilerParams(use_tc_tiling_on_sc=False),
  )(y1, src2d, dst2d)


def _prop2_body(ya, yb, src2d, dst2d, outa, outb,
                isrc, idst, rows, zbuf, acc, sg, ss):
  c = lax.axis_index("c")
  s = lax.axis_index("s")
  rows_per_tile = EROWS // NS
  tile_row0 = s * rows_per_tile
  bufs = (isrc, idst, rows, sg, ss)

  def phase(y_hbm, out_hbm):
    _init_acc(y_hbm, zbuf, acc, s)
    plsc.subcore_barrier()
    _edge_loop(y_hbm, src2d, dst2d, acc, bufs, tile_row0,
               rows_per_tile // ECH)
    plsc.subcore_barrier()
    _drain_acc(acc, zbuf, lambda sl: out_hbm.at[sl], s)

  @pl.when(c == 0)
  def _():
    phase(ya, outa)

  @pl.when(c == 1)
  def _():
    phase(yb, outb)


def _prop2(ya, yb, src2d, dst2d):
  return pl.kernel(
      _prop2_body,
      out_type=[jax.ShapeDtypeStruct((NPAD, 16), jnp.float32)] * 2,
      mesh=_mesh(),
      scratch_types=_PROP_SCRATCH16,
      compiler_params=pltpu.CompilerParams(use_tc_tiling_on_sc=False),
  )(ya, yb, src2d, dst2d)


RT1 = 6272
RT = 7168


def _prep_body(deg_ref, x_ref, dis_ref, y1_ref):
  deg = deg_ref[0] + deg_ref[1] + 1.0
  dis = lax.rsqrt(deg)
  dis_ref[...] = dis
  y1_ref[...] = x_ref[...] * dis


def _tc_prep(degs3, x_p):
  return pl.pallas_call(
      _prep_body,
      grid=(NPAD // RT1,),
      in_specs=[
          pl.BlockSpec((NC, RT1, 1), lambda i: (0, i, 0)),
          pl.BlockSpec((RT1, 16), lambda i: (i, 0)),
      ],
      out_specs=[
          pl.BlockSpec((RT1, 1), lambda i: (i, 0)),
          pl.BlockSpec((RT1, 16), lambda i: (i, 0)),
      ],
      out_shape=[
          jax.ShapeDtypeStruct((NPAD, 1), jnp.float32),
          jax.ShapeDtypeStruct((NPAD, 16), jnp.float32),
      ],
  )(degs3, x_p)


def _layer1_body(dis_ref, z1_ref, w_ref, b_ref, *yq_refs):
  dis = dis_ref[...]
  p1 = (z1_ref[0] + z1_ref[1]) * dis
  h1 = jnp.maximum(
      jnp.dot(p1, w_ref[...], preferred_element_type=jnp.float32)
      + b_ref[...], 0.0)
  y2 = h1 * dis
  for q in range(4):
    yq_refs[q][...] = y2[:, q * 16:(q + 1) * 16]


def _tc_layer1(dis, z1, w1p, b1):
  return pl.pallas_call(
      _layer1_body,
      grid=(NPAD // RT,),
      in_specs=[
          pl.BlockSpec((RT, 1), lambda i: (i, 0)),
          pl.BlockSpec((NC, RT, 16), lambda i: (0, i, 0)),
          pl.BlockSpec((16, 64), lambda i: (0, 0)),
          pl.BlockSpec((1, 64), lambda i: (0, 0)),
      ],
      out_specs=[pl.BlockSpec((RT, 16), lambda i: (i, 0))] * 4,
      out_shape=[jax.ShapeDtypeStruct((NPAD, 16), jnp.float32)] * 4,
  )(dis, z1, w1p, b1)


def _layer2_body(dis_ref, z0_ref, z1_ref, z2_ref, z3_ref, batch_ref,
                 w2_ref, b2_ref, wf1_ref, bf1_ref, wf2_ref, bf2_ref,
                 out_ref, sums_ref, counts_ref):
  i = pl.program_id(0)
  nblk = pl.num_programs(0)
  dis = dis_ref[...]
  p2 = jnp.concatenate(
      [z_ref[...] * dis
       for z_ref in [z0_ref, z1_ref, z2_ref, z3_ref]],
      axis=1)
  h2 = jnp.maximum(
      jnp.dot(p2, w2_ref[...], preferred_element_type=jnp.float32)
      + b2_ref[...], 0.0)
  gid = lax.broadcasted_iota(jnp.int32, (RT, NUM_GRAPHS), 1)
  oh = (batch_ref[...] == gid).astype(jnp.float32)
  blk_sums = lax.dot_general(oh, h2, (((0,), (0,)), ((), ())),
                             preferred_element_type=jnp.float32)
  ones = jnp.ones((RT, 1), jnp.float32)
  blk_counts = lax.dot_general(oh, ones, (((0,), (0,)), ((), ())),
                               preferred_element_type=jnp.float32)

  @pl.when(i == 0)
  def _():
    sums_ref[...] = blk_sums
    counts_ref[...] = blk_counts

  @pl.when(i > 0)
  def _():
    sums_ref[...] += blk_sums
    counts_ref[...] += blk_counts

  @pl.when(i == nblk - 1)
  def _():
    g = sums_ref[...] / jnp.maximum(counts_ref[...], 1.0)
    o1 = jnp.maximum(
        jnp.dot(g, wf1_ref[...], preferred_element_type=jnp.float32)
        + bf1_ref[...], 0.0)
    out_ref[...] = (
        jnp.dot(o1, wf2_ref[...], preferred_element_type=jnp.float32)
        + bf2_ref[...])


def _tc_layer2(dis, zq, batch2d, w2, b2, wf1, bf1, wf2, bf2):
  row = lambda i: (i, 0)
  full = lambda i: (0, 0)
  return pl.pallas_call(
      _layer2_body,
      grid=(NPAD // RT,),
      in_specs=[
          pl.BlockSpec((RT, 1), row),
          pl.BlockSpec((RT, 16), row),
          pl.BlockSpec((RT, 16), row),
          pl.BlockSpec((RT, 16), row),
          pl.BlockSpec((RT, 16), row),
          pl.BlockSpec((RT, 1), row),
          pl.BlockSpec((64, 128), full),
          pl.BlockSpec((1, 128), full),
          pl.BlockSpec((128, 64), full),
          pl.BlockSpec((1, 64), full),
          pl.BlockSpec((64, 1), full),
          pl.BlockSpec((1, 1), full),
      ],
      out_specs=pl.BlockSpec((NUM_GRAPHS, 1), full),
      out_shape=jax.ShapeDtypeStruct((NUM_GRAPHS, 1), jnp.float32),
      scratch_shapes=[
          pltpu.VMEM((NUM_GRAPHS, 128), jnp.float32),
          pltpu.VMEM((NUM_GRAPHS, 1), jnp.float32),
      ],
  )(dis, *zq, batch2d, w2, b2, wf1, bf1, wf2, bf2)


@jax.jit
def kernel(x, edge_index, batch, W1, b1, W2, b2, Wf1, bf1, Wf2, bf2):
  src = edge_index[0].astype(jnp.int32)
  dst = edge_index[1].astype(jnp.int32)
  epad = N_NODES + jnp.arange(EPAD - N_EDGES, dtype=jnp.int32) % (NPAD - N_NODES)
  src2d = jnp.concatenate([src, epad]).reshape(EROWS, 128)
  dst2d = jnp.concatenate([dst, epad]).reshape(EROWS, 128)
  batch2d = jnp.concatenate(
      [batch.astype(jnp.int32),
       jnp.full((NPAD - N_NODES,), NUM_GRAPHS, jnp.int32)]).reshape(NPAD, 1)
  x_p = jnp.zeros((NPAD, 16), jnp.float32).at[:N_NODES, :3].set(x)
  w1p = jnp.zeros((16, 64), jnp.float32).at[:3].set(W1)

  degA, degB = _hist(dst2d)
  dis, y1 = _tc_prep(jnp.stack([degA, degB]).reshape(NC, NPAD, 1), x_p)
  z1 = _prop1(y1, src2d, dst2d)
  yq = _tc_layer1(dis, z1, w1p, b1.reshape(1, 64))
  z0, z1b = _prop2(yq[0], yq[1], src2d, dst2d)
  z2, z3 = _prop2(yq[2], yq[3], src2d, dst2d)
  out = _tc_layer2(dis, (z0, z1b, z2, z3), batch2d,
                   W2, b2.reshape(1, 128), Wf1, bf1.reshape(1, 64),
                   Wf2, bf2.reshape(1, 1))
  return out
